# baseline (speedup 1.0000x reference)
import jax
import jax.numpy as jnp
from jax.experimental import pallas as pl

N_CODE = 1024
DIM = 64
TOK = 1024
NB = 16
GB = 4
T = GB * TOK


def _vq_body(z_ref, w_ref, zq_ref, idx_ref, loss_ref):
    b = pl.program_id(0)
    w = w_ref[...]
    zc = jnp.concatenate([z_ref[k] for k in range(GB)], axis=1)
    zn = jnp.sum(zc * zc, axis=0, keepdims=True)
    wn = jnp.sum(w * w, axis=1, keepdims=True)
    mm2 = jax.lax.dot_general(w + w, zc, (((1,), (0,)), ((), ())),
                              preferred_element_type=jnp.float32)
    dist = (zn + wn) - mm2
    m = jnp.min(dist, axis=0, keepdims=True)
    jf = jax.lax.broadcasted_iota(jnp.int32, (N_CODE, 1), 0).astype(jnp.float32)
    cand = (dist - m) * jnp.float32(2.0 ** 37) + jf
    idx_f = jnp.min(cand, axis=0)
    idx = idx_f.astype(jnp.int32)
    e = (cand == idx_f[None, :]).astype(jnp.bfloat16)
    zq_t = jax.lax.dot_general(w.astype(jnp.bfloat16), e,
                               (((0,), (0,)), ((), ())),
                               preferred_element_type=jnp.float32)
    for k in range(GB):
        idx_ref[k, 0, :] = idx[k * TOK:(k + 1) * TOK]
        zq_ref[k] = zq_t[:, k * TOK:(k + 1) * TOK]
    part = jnp.sum(m, axis=(0, 1), keepdims=True)

    @pl.when(b == 0)
    def _init():
        loss_ref[...] = jnp.zeros((1, 1), jnp.float32)

    loss_ref[...] += part

    @pl.when(b == NB // GB - 1)
    def _fin():
        loss_ref[...] = loss_ref[...] / (NB * TOK * DIM)


def kernel(z, W):
    B, C, H, Wd = z.shape
    z3 = z.reshape(B, C, H * Wd)
    zq3, idx3, loss = pl.pallas_call(
        _vq_body,
        grid=(B // GB,),
        in_specs=[
            pl.BlockSpec((GB, C, H * Wd), lambda b: (b, 0, 0)),
            pl.BlockSpec((N_CODE, DIM), lambda b: (0, 0)),
        ],
        out_specs=[
            pl.BlockSpec((GB, C, H * Wd), lambda b: (b, 0, 0)),
            pl.BlockSpec((GB, 1, H * Wd), lambda b: (b, 0, 0)),
            pl.BlockSpec((1, 1), lambda b: (0, 0)),
        ],
        out_shape=[
            jax.ShapeDtypeStruct((B, C, H * Wd), jnp.float32),
            jax.ShapeDtypeStruct((B, 1, H * Wd), jnp.int32),
            jax.ShapeDtypeStruct((1, 1), jnp.float32),
        ],
    )(z3, W)
    z_q = zq3.reshape(B, C, H, Wd)
    codebook_loss = loss.reshape(())
    commitment_loss = 0.25 * codebook_loss
    min_encoding_indices = idx3.reshape(B, H, Wd)
    return (z_q, codebook_loss, commitment_loss, min_encoding_indices)


# speedup vs baseline: 1.0199x; 1.0199x over previous
import jax
import jax.numpy as jnp
from jax.experimental import pallas as pl

N_CODE = 1024
DIM = 64
TOK = 1024
NB = 16
GB = 4
T = GB * TOK


def _vq_body(z_ref, w_ref, zq_ref, idx_ref, loss_ref):
    b = pl.program_id(0)
    w = w_ref[...]
    zc = jnp.concatenate([z_ref[k] for k in range(GB)], axis=1)
    zn = jnp.sum(zc * zc, axis=0, keepdims=True)
    wn = jnp.sum(w * w, axis=1, keepdims=True)
    mm2 = jax.lax.dot_general(w + w, zc, (((1,), (0,)), ((), ())),
                              preferred_element_type=jnp.float32)
    dist = (zn + wn) - mm2
    m = jnp.min(dist, axis=0, keepdims=True)
    jf = jax.lax.broadcasted_iota(jnp.int32, (N_CODE, 1), 0).astype(jnp.float32)
    idx_f = jnp.min((dist - m) * jnp.float32(2.0 ** 37) + jf, axis=0)
    idx = idx_f.astype(jnp.int32)
    e = (jax.lax.broadcasted_iota(jnp.int32, (N_CODE, T), 0)
         == idx[None, :]).astype(jnp.bfloat16)
    zq_t = jax.lax.dot_general(w.astype(jnp.bfloat16), e,
                               (((0,), (0,)), ((), ())),
                               preferred_element_type=jnp.float32)
    for k in range(GB):
        idx_ref[k, 0, :] = idx[k * TOK:(k + 1) * TOK]
        zq_ref[k] = zq_t[:, k * TOK:(k + 1) * TOK]
    part = jnp.sum(m, axis=(0, 1), keepdims=True)

    @pl.when(b == 0)
    def _init():
        loss_ref[...] = jnp.zeros((1, 1), jnp.float32)

    loss_ref[...] += part

    @pl.when(b == NB // GB - 1)
    def _fin():
        loss_ref[...] = loss_ref[...] / (NB * TOK * DIM)


def kernel(z, W):
    B, C, H, Wd = z.shape
    z3 = z.reshape(B, C, H * Wd)
    zq3, idx3, loss = pl.pallas_call(
        _vq_body,
        grid=(B // GB,),
        in_specs=[
            pl.BlockSpec((GB, C, H * Wd), lambda b: (b, 0, 0)),
            pl.BlockSpec((N_CODE, DIM), lambda b: (0, 0)),
        ],
        out_specs=[
            pl.BlockSpec((GB, C, H * Wd), lambda b: (b, 0, 0)),
            pl.BlockSpec((GB, 1, H * Wd), lambda b: (b, 0, 0)),
            pl.BlockSpec((1, 1), lambda b: (0, 0)),
        ],
        out_shape=[
            jax.ShapeDtypeStruct((B, C, H * Wd), jnp.float32),
            jax.ShapeDtypeStruct((B, 1, H * Wd), jnp.int32),
            jax.ShapeDtypeStruct((1, 1), jnp.float32),
        ],
    )(z3, W)
    z_q = zq3.reshape(B, C, H, Wd)
    codebook_loss = loss.reshape(())
    commitment_loss = 0.25 * codebook_loss
    min_encoding_indices = idx3.reshape(B, H, Wd)
    return (z_q, codebook_loss, commitment_loss, min_encoding_indices)
